# baseline (speedup 1.0000x reference)
import functools

import jax
import jax.numpy as jnp
from jax import lax
from jax.experimental import pallas as pl
from jax.experimental.pallas import tpu as pltpu
from jax.experimental.pallas import tpu_sc as plsc

D_MODEL = 768
NUM_EXPERTS = 8
TOP_K = 2
MASK_W = TOP_K * NUM_EXPERTS


_TB = 1024
_NBUF = 6


def _logits_body(x_hbm, w_ref, out_ref, *scratch):
    n = x_hbm.shape[0]
    nblk = n // _TB
    bufs = scratch[:_NBUF]
    sems = scratch[_NBUF]

    def start(i):
        pltpu.make_async_copy(
            x_hbm.at[pl.ds(i * _TB, _TB), :],
            bufs[i % _NBUF], sems.at[i % _NBUF]).start()

    for i in range(_NBUF):
        start(i)
    w = w_ref[...]
    for i in range(nblk):
        b = i % _NBUF
        pltpu.make_async_copy(
            x_hbm.at[pl.ds(i * _TB, _TB), :], bufs[b], sems.at[b]).wait()
        r = lax.dot_general(
            w, bufs[b][...],
            dimension_numbers=(((1,), (1,)), ((), ())),
            preferred_element_type=jnp.float32)
        for j in range(NUM_EXPERTS):
            out_ref[pl.ds(j * n + i * _TB, _TB)] = r[j]
        if i + _NBUF < nblk:
            start(i + _NBUF)


def _compute_logits_t(x, W):
    n = x.shape[0]
    return pl.pallas_call(
        _logits_body,
        in_specs=[pl.BlockSpec(memory_space=pltpu.HBM),
                  pl.BlockSpec((NUM_EXPERTS, D_MODEL), lambda: (0, 0))],
        out_specs=pl.BlockSpec((NUM_EXPERTS * n,), lambda: (0,)),
        out_shape=jax.ShapeDtypeStruct((NUM_EXPERTS * n,), jnp.float32),
        scratch_shapes=(
            [pltpu.VMEM((_TB, D_MODEL), jnp.float32) for _ in range(_NBUF)]
            + [pltpu.SemaphoreType.DMA((_NBUF,))]
        ),
    )(x, W)


@functools.lru_cache(maxsize=None)
def _make_router(n):
    info = plsc.get_sparse_core_info()
    nc, ns, lanes = info.num_cores, info.num_subcores, info.num_lanes
    nw = nc * ns
    tpw = n // nw
    mesh = plsc.VectorSubcoreMesh(core_axis_name="c", subcore_axis_name="s")

    @functools.partial(
        pl.kernel, mesh=mesh,
        compiler_params=pltpu.CompilerParams(
            needs_layout_passes=False, use_tc_tiling_on_sc=False),
        out_type=[
            jax.ShapeDtypeStruct((TOP_K * n,), jnp.float32),
            jax.ShapeDtypeStruct((TOP_K * n,), jnp.int32),
            jax.ShapeDtypeStruct((MASK_W * n,), jnp.float32),
        ],
        scratch_types=(
            [pltpu.VMEM((tpw,), jnp.float32) for _ in range(NUM_EXPERTS)]
            + [pltpu.VMEM((tpw,), jnp.float32) for _ in range(TOP_K)]
            + [pltpu.VMEM((tpw,), jnp.int32) for _ in range(TOP_K)]
            + [pltpu.VMEM((tpw,), jnp.float32) for _ in range(MASK_W)]
            + [pltpu.SemaphoreType.DMA]
        ),
    )
    def router(logits_hbm, probs_hbm, idx_hbm, mask_hbm, *scratch):
        e_v = scratch[0:NUM_EXPERTS]
        p_v = scratch[NUM_EXPERTS:NUM_EXPERTS + TOP_K]
        ix_v = scratch[NUM_EXPERTS + TOP_K:NUM_EXPERTS + 2 * TOP_K]
        m_v = scratch[NUM_EXPERTS + 2 * TOP_K:NUM_EXPERTS + 2 * TOP_K + MASK_W]
        sem = scratch[-1]
        wid = lax.axis_index("s") * nc + lax.axis_index("c")
        base = wid * tpw
        copies = [
            pltpu.async_copy(
                logits_hbm.at[pl.ds(j * n + base, tpw)], e_v[j], sem)
            for j in range(NUM_EXPERTS)]
        for c in copies:
            c.wait()

        @plsc.parallel_loop(0, tpw // lanes, unroll=2)
        def chunk(i):
            sl = pl.ds(i * lanes, lanes)
            e = [e_v[j][sl] for j in range(NUM_EXPERTS)]
            m1 = e[0]
            i1 = jnp.zeros((lanes,), jnp.int32)
            for j in range(1, NUM_EXPERTS):
                gt = e[j] > m1
                m1 = jnp.where(gt, e[j], m1)
                i1 = jnp.where(gt, j, i1)
            m2 = jnp.full((lanes,), -3e38, jnp.float32)
            i2 = jnp.zeros((lanes,), jnp.int32)
            for j in range(NUM_EXPERTS):
                gt = (e[j] > m2) & (i1 != j)
                m2 = jnp.where(gt, e[j], m2)
                i2 = jnp.where(gt, j, i2)
            d = jnp.exp(m2 - m1)
            p1 = 1.0 / (1.0 + d)
            p2 = d * p1
            p_v[0][sl] = p1
            p_v[1][sl] = p2
            ix_v[0][sl] = i1
            ix_v[1][sl] = i2
            for k in range(NUM_EXPERTS):
                m_v[k][sl] = jnp.where(i1 == k, 1.0, 0.0)
                m_v[NUM_EXPERTS + k][sl] = jnp.where(i2 == k, 1.0, 0.0)

        out = []
        for r in range(TOP_K):
            out.append(pltpu.async_copy(
                p_v[r], probs_hbm.at[pl.ds(r * n + base, tpw)], sem))
            out.append(pltpu.async_copy(
                ix_v[r], idx_hbm.at[pl.ds(r * n + base, tpw)], sem))
        for k in range(MASK_W):
            out.append(pltpu.async_copy(
                m_v[k], mask_hbm.at[pl.ds(k * n + base, tpw)], sem))
        for c in out:
            c.wait()

    return router


_NSLAB = 2


def kernel(x, W):
    n = x.shape[0]
    ns = n // _NSLAB
    router = _make_router(ns)
    parts = []
    for s in range(_NSLAB):
        logits_t = _compute_logits_t(x[s * ns:(s + 1) * ns], W)
        parts.append(router(logits_t))
    probs = jnp.concatenate(
        [p.reshape(TOP_K, ns) for p, _, _ in parts], axis=1).T
    idx = jnp.concatenate(
        [i.reshape(TOP_K, ns) for _, i, _ in parts], axis=1).T
    mask = jnp.concatenate(
        [m.reshape(TOP_K, NUM_EXPERTS, ns) for _, _, m in parts],
        axis=2).transpose(2, 0, 1)
    return probs, idx, mask


# speedup vs baseline: 1.9755x; 1.9755x over previous
import functools

import jax
import jax.numpy as jnp
from jax import lax
from jax.experimental import pallas as pl
from jax.experimental.pallas import tpu as pltpu
from jax.experimental.pallas import tpu_sc as plsc

D_MODEL = 768
NUM_EXPERTS = 8
TOP_K = 2
MASK_W = TOP_K * NUM_EXPERTS


_TB = 1024
_NBUF = 8


def _logits_body(x_hbm, w_ref, out_ref, *scratch):
    n = x_hbm.shape[0]
    nblk = n // _TB
    bufs = scratch[:_NBUF]
    sems = scratch[_NBUF]

    def start(i):
        pltpu.make_async_copy(
            x_hbm.at[pl.ds(i * _TB, _TB), :],
            bufs[i % _NBUF], sems.at[i % _NBUF]).start()

    for i in range(_NBUF):
        start(i)
    w = w_ref[...]
    for i in range(nblk):
        b = i % _NBUF
        pltpu.make_async_copy(
            x_hbm.at[pl.ds(i * _TB, _TB), :], bufs[b], sems.at[b]).wait()
        r = lax.dot_general(
            w, bufs[b][...],
            dimension_numbers=(((1,), (1,)), ((), ())),
            preferred_element_type=jnp.float32)
        for j in range(NUM_EXPERTS):
            out_ref[pl.ds(j * n + i * _TB, _TB)] = r[j]
        if i + _NBUF < nblk:
            start(i + _NBUF)


def _compute_logits_t(x, W):
    n = x.shape[0]
    return pl.pallas_call(
        _logits_body,
        in_specs=[pl.BlockSpec(memory_space=pltpu.HBM),
                  pl.BlockSpec((NUM_EXPERTS, D_MODEL), lambda: (0, 0))],
        out_specs=pl.BlockSpec((NUM_EXPERTS * n,), lambda: (0,)),
        out_shape=jax.ShapeDtypeStruct((NUM_EXPERTS * n,), jnp.float32),
        scratch_shapes=(
            [pltpu.VMEM((_TB, D_MODEL), jnp.float32) for _ in range(_NBUF)]
            + [pltpu.SemaphoreType.DMA((_NBUF,))]
        ),
    )(x, W)


@functools.lru_cache(maxsize=None)
def _make_router(n):
    info = plsc.get_sparse_core_info()
    nc, ns, lanes = info.num_cores, info.num_subcores, info.num_lanes
    nw = nc * ns
    tpw = n // nw
    mesh = plsc.VectorSubcoreMesh(core_axis_name="c", subcore_axis_name="s")

    @functools.partial(
        pl.kernel, mesh=mesh,
        compiler_params=pltpu.CompilerParams(
            needs_layout_passes=False, use_tc_tiling_on_sc=False),
        out_type=[
            jax.ShapeDtypeStruct((TOP_K * n,), jnp.float32),
            jax.ShapeDtypeStruct((TOP_K * n,), jnp.int32),
            jax.ShapeDtypeStruct((MASK_W * n,), jnp.float32),
        ],
        scratch_types=(
            [pltpu.VMEM((tpw,), jnp.float32) for _ in range(NUM_EXPERTS)]
            + [pltpu.VMEM((tpw,), jnp.float32) for _ in range(TOP_K)]
            + [pltpu.VMEM((tpw,), jnp.int32) for _ in range(TOP_K)]
            + [pltpu.VMEM((tpw,), jnp.float32) for _ in range(MASK_W)]
            + [pltpu.SemaphoreType.DMA]
        ),
    )
    def router(logits_hbm, probs_hbm, idx_hbm, mask_hbm, *scratch):
        e_v = scratch[0:NUM_EXPERTS]
        p_v = scratch[NUM_EXPERTS:NUM_EXPERTS + TOP_K]
        ix_v = scratch[NUM_EXPERTS + TOP_K:NUM_EXPERTS + 2 * TOP_K]
        m_v = scratch[NUM_EXPERTS + 2 * TOP_K:NUM_EXPERTS + 2 * TOP_K + MASK_W]
        sem = scratch[-1]
        wid = lax.axis_index("s") * nc + lax.axis_index("c")
        base = wid * tpw
        copies = [
            pltpu.async_copy(
                logits_hbm.at[pl.ds(j * n + base, tpw)], e_v[j], sem)
            for j in range(NUM_EXPERTS)]
        for c in copies:
            c.wait()

        @plsc.parallel_loop(0, tpw // lanes, unroll=2)
        def chunk(i):
            sl = pl.ds(i * lanes, lanes)
            e = [e_v[j][sl] for j in range(NUM_EXPERTS)]
            m1 = e[0]
            i1 = jnp.zeros((lanes,), jnp.int32)
            for j in range(1, NUM_EXPERTS):
                gt = e[j] > m1
                m1 = jnp.where(gt, e[j], m1)
                i1 = jnp.where(gt, j, i1)
            m2 = jnp.full((lanes,), -3e38, jnp.float32)
            i2 = jnp.zeros((lanes,), jnp.int32)
            for j in range(NUM_EXPERTS):
                gt = (e[j] > m2) & (i1 != j)
                m2 = jnp.where(gt, e[j], m2)
                i2 = jnp.where(gt, j, i2)
            d = jnp.exp(m2 - m1)
            p1 = 1.0 / (1.0 + d)
            p2 = d * p1
            p_v[0][sl] = p1
            p_v[1][sl] = p2
            ix_v[0][sl] = i1
            ix_v[1][sl] = i2
            for k in range(NUM_EXPERTS):
                m_v[k][sl] = jnp.where(i1 == k, 1.0, 0.0)
                m_v[NUM_EXPERTS + k][sl] = jnp.where(i2 == k, 1.0, 0.0)

        out = []
        for r in range(TOP_K):
            out.append(pltpu.async_copy(
                p_v[r], probs_hbm.at[pl.ds(r * n + base, tpw)], sem))
            out.append(pltpu.async_copy(
                ix_v[r], idx_hbm.at[pl.ds(r * n + base, tpw)], sem))
        for k in range(MASK_W):
            out.append(pltpu.async_copy(
                m_v[k], mask_hbm.at[pl.ds(k * n + base, tpw)], sem))
        for c in out:
            c.wait()

    return router


def kernel(x, W):
    n = x.shape[0]
    logits_t = _compute_logits_t(x, W)
    probs_t, idx_t, mask_t = _make_router(n)(logits_t)
    probs = probs_t.reshape(TOP_K, n).T
    idx = idx_t.reshape(TOP_K, n).T
    mask = mask_t.reshape(TOP_K, NUM_EXPERTS, n).transpose(2, 0, 1)
    return probs, idx, mask


# speedup vs baseline: 2.2482x; 1.1380x over previous
import functools

import jax
import jax.numpy as jnp
from jax import lax
from jax.experimental import pallas as pl
from jax.experimental.pallas import tpu as pltpu
from jax.experimental.pallas import tpu_sc as plsc

D_MODEL = 768
NUM_EXPERTS = 8
TOP_K = 2
MASK_W = TOP_K * NUM_EXPERTS


_TB = 1024
_NBUF = 8


def _logits_body(x_hbm, w_ref, out_ref, *scratch):
    n = x_hbm.shape[0]
    nblk = n // _TB
    bufs = scratch[:_NBUF]
    sems = scratch[_NBUF]

    def start(i):
        pltpu.make_async_copy(
            x_hbm.at[pl.ds(i * _TB, _TB), :],
            bufs[i % _NBUF], sems.at[i % _NBUF]).start()

    for i in range(_NBUF):
        start(i)
    w = w_ref[...]
    for i in range(nblk):
        b = i % _NBUF
        pltpu.make_async_copy(
            x_hbm.at[pl.ds(i * _TB, _TB), :], bufs[b], sems.at[b]).wait()
        r = lax.dot_general(
            w, bufs[b][...],
            dimension_numbers=(((1,), (1,)), ((), ())),
            preferred_element_type=jnp.float32)
        for j in range(NUM_EXPERTS):
            out_ref[pl.ds(j * n + i * _TB, _TB)] = r[j]
        if i + _NBUF < nblk:
            start(i + _NBUF)


def _compute_logits_t(x, W):
    n = x.shape[0]
    return pl.pallas_call(
        _logits_body,
        in_specs=[pl.BlockSpec(memory_space=pltpu.HBM),
                  pl.BlockSpec((NUM_EXPERTS, D_MODEL), lambda: (0, 0))],
        out_specs=pl.BlockSpec((NUM_EXPERTS * n,), lambda: (0,)),
        out_shape=jax.ShapeDtypeStruct((NUM_EXPERTS * n,), jnp.float32),
        scratch_shapes=(
            [pltpu.VMEM((_TB, D_MODEL), jnp.float32) for _ in range(_NBUF)]
            + [pltpu.SemaphoreType.DMA((_NBUF,))]
        ),
    )(x, W)


@functools.lru_cache(maxsize=None)
def _make_router(n):
    info = plsc.get_sparse_core_info()
    nc, ns, lanes = info.num_cores, info.num_subcores, info.num_lanes
    nw = nc * ns
    tpw = n // nw
    mesh = plsc.VectorSubcoreMesh(core_axis_name="c", subcore_axis_name="s")

    @functools.partial(
        pl.kernel, mesh=mesh,
        compiler_params=pltpu.CompilerParams(
            needs_layout_passes=False, use_tc_tiling_on_sc=False),
        out_type=[
            jax.ShapeDtypeStruct((TOP_K * n,), jnp.float32),
            jax.ShapeDtypeStruct((TOP_K * n,), jnp.int32),
            jax.ShapeDtypeStruct((MASK_W * n,), jnp.float32),
        ],
        scratch_types=(
            [pltpu.VMEM((tpw,), jnp.float32) for _ in range(NUM_EXPERTS)]
            + [pltpu.VMEM((TOP_K * tpw,), jnp.float32)]
            + [pltpu.VMEM((TOP_K * tpw,), jnp.int32)]
            + [pltpu.VMEM((NUM_EXPERTS * tpw,), jnp.float32) for _ in range(TOP_K)]
            + [pltpu.SemaphoreType.DMA]
        ),
    )
    def router(logits_hbm, probs_hbm, idx_hbm, mask_hbm, *scratch):
        e_v = scratch[0:NUM_EXPERTS]
        p_v = scratch[NUM_EXPERTS]
        ix_v = scratch[NUM_EXPERTS + 1]
        m_v = scratch[NUM_EXPERTS + 2:NUM_EXPERTS + 2 + TOP_K]
        sem = scratch[-1]
        wid = lax.axis_index("s") * nc + lax.axis_index("c")
        base = wid * tpw
        copies = [
            pltpu.async_copy(
                logits_hbm.at[pl.ds(j * n + base, tpw)], e_v[j], sem)
            for j in range(NUM_EXPERTS)]
        for c in copies:
            c.wait()

        @plsc.parallel_loop(0, tpw // lanes, unroll=2)
        def chunk(i):
            sl = pl.ds(i * lanes, lanes)
            e = [e_v[j][sl] for j in range(NUM_EXPERTS)]
            m1 = e[0]
            i1 = jnp.zeros((lanes,), jnp.int32)
            for j in range(1, NUM_EXPERTS):
                gt = e[j] > m1
                m1 = jnp.where(gt, e[j], m1)
                i1 = jnp.where(gt, j, i1)
            m2 = jnp.full((lanes,), -3e38, jnp.float32)
            i2 = jnp.zeros((lanes,), jnp.int32)
            for j in range(NUM_EXPERTS):
                gt = (e[j] > m2) & (i1 != j)
                m2 = jnp.where(gt, e[j], m2)
                i2 = jnp.where(gt, j, i2)
            d = jnp.exp(m2 - m1)
            p1 = 1.0 / (1.0 + d)
            p2 = d * p1
            lb = (i // 8) * 256
            lo = (i % 8) * lanes
            p_v[pl.ds(lb + lo, lanes)] = p1
            p_v[pl.ds(lb + 128 + lo, lanes)] = p2
            ix_v[pl.ds(lb + lo, lanes)] = i1
            ix_v[pl.ds(lb + 128 + lo, lanes)] = i2
            mb = (i // 8) * 1024
            for k in range(NUM_EXPERTS):
                m_v[0][pl.ds(mb + k * 128 + lo, lanes)] = jnp.where(i1 == k, 1.0, 0.0)
                m_v[1][pl.ds(mb + k * 128 + lo, lanes)] = jnp.where(i2 == k, 1.0, 0.0)

        out = [
            pltpu.async_copy(
                p_v, probs_hbm.at[pl.ds(TOP_K * base, TOP_K * tpw)], sem),
            pltpu.async_copy(
                ix_v, idx_hbm.at[pl.ds(TOP_K * base, TOP_K * tpw)], sem),
        ]
        for r in range(TOP_K):
            out.append(pltpu.async_copy(
                m_v[r],
                mask_hbm.at[pl.ds(r * NUM_EXPERTS * n + NUM_EXPERTS * base,
                                  NUM_EXPERTS * tpw)], sem))
        for c in out:
            c.wait()

    return router


def kernel(x, W):
    n = x.shape[0]
    nb = n // 128
    logits_t = _compute_logits_t(x, W)
    probs_t, idx_t, mask_t = _make_router(n)(logits_t)
    probs = probs_t.reshape(nb, TOP_K, 128).transpose(0, 2, 1).reshape(n, TOP_K)
    idx = idx_t.reshape(nb, TOP_K, 128).transpose(0, 2, 1).reshape(n, TOP_K)
    mask = mask_t.reshape(TOP_K, nb, NUM_EXPERTS, 128).transpose(
        1, 3, 0, 2).reshape(n, TOP_K, NUM_EXPERTS)
    return probs, idx, mask


# speedup vs baseline: 2.2530x; 1.0022x over previous
import functools

import jax
import jax.numpy as jnp
from jax import lax
from jax.experimental import pallas as pl
from jax.experimental.pallas import tpu as pltpu
from jax.experimental.pallas import tpu_sc as plsc

D_MODEL = 768
NUM_EXPERTS = 8
TOP_K = 2
MASK_W = TOP_K * NUM_EXPERTS


_TB = 1024
_NBUF = 8


def _logits_body(x_hbm, w_ref, out_ref, *scratch):
    n = x_hbm.shape[0]
    nblk = n // _TB
    bufs = scratch[:_NBUF]
    sems = scratch[_NBUF]

    def start(i):
        pltpu.make_async_copy(
            x_hbm.at[pl.ds(i * _TB, _TB), :],
            bufs[i % _NBUF], sems.at[i % _NBUF]).start()

    for i in range(_NBUF):
        start(i)
    w = w_ref[...]
    for i in range(nblk):
        b = i % _NBUF
        pltpu.make_async_copy(
            x_hbm.at[pl.ds(i * _TB, _TB), :], bufs[b], sems.at[b]).wait()
        r = lax.dot_general(
            w, bufs[b][...],
            dimension_numbers=(((1,), (1,)), ((), ())),
            preferred_element_type=jnp.float32)
        for j in range(NUM_EXPERTS):
            out_ref[pl.ds(j * n + i * _TB, _TB)] = r[j]
        if i + _NBUF < nblk:
            start(i + _NBUF)


def _compute_logits_t(x, W):
    n = x.shape[0]
    return pl.pallas_call(
        _logits_body,
        in_specs=[pl.BlockSpec(memory_space=pltpu.HBM),
                  pl.BlockSpec((NUM_EXPERTS, D_MODEL), lambda: (0, 0))],
        out_specs=pl.BlockSpec((NUM_EXPERTS * n,), lambda: (0,)),
        out_shape=jax.ShapeDtypeStruct((NUM_EXPERTS * n,), jnp.float32),
        scratch_shapes=(
            [pltpu.VMEM((_TB, D_MODEL), jnp.float32) for _ in range(_NBUF)]
            + [pltpu.SemaphoreType.DMA((_NBUF,))]
        ),
    )(x, W)


@functools.lru_cache(maxsize=None)
def _make_router(n):
    info = plsc.get_sparse_core_info()
    nc, ns, lanes = info.num_cores, info.num_subcores, info.num_lanes
    nw = nc * ns
    tpw = n // nw
    mesh = plsc.VectorSubcoreMesh(core_axis_name="c", subcore_axis_name="s")

    @functools.partial(
        pl.kernel, mesh=mesh,
        compiler_params=pltpu.CompilerParams(
            needs_layout_passes=False, use_tc_tiling_on_sc=False),
        out_type=[
            jax.ShapeDtypeStruct((TOP_K * n,), jnp.float32),
            jax.ShapeDtypeStruct((TOP_K * n,), jnp.int32),
            jax.ShapeDtypeStruct((MASK_W * n,), jnp.float32),
        ],
        scratch_types=(
            [pltpu.VMEM((tpw,), jnp.float32) for _ in range(NUM_EXPERTS)]
            + [pltpu.VMEM((TOP_K * tpw,), jnp.float32)]
            + [pltpu.VMEM((TOP_K * tpw,), jnp.int32)]
            + [pltpu.VMEM((NUM_EXPERTS * tpw,), jnp.float32) for _ in range(TOP_K)]
            + [pltpu.SemaphoreType.DMA]
        ),
    )
    def router(logits_hbm, probs_hbm, idx_hbm, mask_hbm, *scratch):
        e_v = scratch[0:NUM_EXPERTS]
        p_v = scratch[NUM_EXPERTS]
        ix_v = scratch[NUM_EXPERTS + 1]
        m_v = scratch[NUM_EXPERTS + 2:NUM_EXPERTS + 2 + TOP_K]
        sem = scratch[-1]
        wid = lax.axis_index("s") * nc + lax.axis_index("c")
        base = wid * tpw
        copies = [
            pltpu.async_copy(
                logits_hbm.at[pl.ds(j * n + base, tpw)], e_v[j], sem)
            for j in range(NUM_EXPERTS)]
        for c in copies:
            c.wait()

        @plsc.parallel_loop(0, tpw // lanes, unroll=4)
        def chunk(i):
            sl = pl.ds(i * lanes, lanes)
            e = [e_v[j][sl] for j in range(NUM_EXPERTS)]
            m1 = e[0]
            i1 = jnp.zeros((lanes,), jnp.int32)
            for j in range(1, NUM_EXPERTS):
                gt = e[j] > m1
                m1 = jnp.where(gt, e[j], m1)
                i1 = jnp.where(gt, j, i1)
            m2 = jnp.full((lanes,), -3e38, jnp.float32)
            i2 = jnp.zeros((lanes,), jnp.int32)
            for j in range(NUM_EXPERTS):
                gt = (e[j] > m2) & (i1 != j)
                m2 = jnp.where(gt, e[j], m2)
                i2 = jnp.where(gt, j, i2)
            d = jnp.exp(m2 - m1)
            p1 = 1.0 / (1.0 + d)
            p2 = d * p1
            lb = (i // 8) * 256
            lo = (i % 8) * lanes
            p_v[pl.ds(lb + lo, lanes)] = p1
            p_v[pl.ds(lb + 128 + lo, lanes)] = p2
            ix_v[pl.ds(lb + lo, lanes)] = i1
            ix_v[pl.ds(lb + 128 + lo, lanes)] = i2
            mb = (i // 8) * 1024
            for k in range(NUM_EXPERTS):
                m_v[0][pl.ds(mb + k * 128 + lo, lanes)] = jnp.where(i1 == k, 1.0, 0.0)
                m_v[1][pl.ds(mb + k * 128 + lo, lanes)] = jnp.where(i2 == k, 1.0, 0.0)

        out = [
            pltpu.async_copy(
                p_v, probs_hbm.at[pl.ds(TOP_K * base, TOP_K * tpw)], sem),
            pltpu.async_copy(
                ix_v, idx_hbm.at[pl.ds(TOP_K * base, TOP_K * tpw)], sem),
        ]
        for r in range(TOP_K):
            out.append(pltpu.async_copy(
                m_v[r],
                mask_hbm.at[pl.ds(r * NUM_EXPERTS * n + NUM_EXPERTS * base,
                                  NUM_EXPERTS * tpw)], sem))
        for c in out:
            c.wait()

    return router


def kernel(x, W):
    n = x.shape[0]
    nb = n // 128
    logits_t = _compute_logits_t(x, W)
    probs_t, idx_t, mask_t = _make_router(n)(logits_t)
    probs = probs_t.reshape(nb, TOP_K, 128).transpose(0, 2, 1).reshape(n, TOP_K)
    idx = idx_t.reshape(nb, TOP_K, 128).transpose(0, 2, 1).reshape(n, TOP_K)
    mask = mask_t.reshape(TOP_K, nb, NUM_EXPERTS, 128).transpose(
        1, 3, 0, 2).reshape(n, TOP_K, NUM_EXPERTS)
    return probs, idx, mask


# speedup vs baseline: 2.2568x; 1.0017x over previous
import functools

import jax
import jax.numpy as jnp
from jax import lax
from jax.experimental import pallas as pl
from jax.experimental.pallas import tpu as pltpu
from jax.experimental.pallas import tpu_sc as plsc

D_MODEL = 768
NUM_EXPERTS = 8
TOP_K = 2
MASK_W = TOP_K * NUM_EXPERTS


_TB = 1024
_NBUF = 8


def _logits_body(x_hbm, w_ref, out_ref, *scratch):
    n = x_hbm.shape[0]
    nblk = n // _TB
    bufs = scratch[:_NBUF]
    sems = scratch[_NBUF]

    def start(i):
        pltpu.make_async_copy(
            x_hbm.at[pl.ds(i * _TB, _TB), :],
            bufs[i % _NBUF], sems.at[i % _NBUF]).start()

    for i in range(_NBUF):
        start(i)
    w = w_ref[...]
    for i in range(nblk):
        b = i % _NBUF
        pltpu.make_async_copy(
            x_hbm.at[pl.ds(i * _TB, _TB), :], bufs[b], sems.at[b]).wait()
        r = lax.dot_general(
            w, bufs[b][...],
            dimension_numbers=(((1,), (1,)), ((), ())),
            preferred_element_type=jnp.float32)
        for j in range(NUM_EXPERTS):
            out_ref[pl.ds(i * NUM_EXPERTS * _TB + j * _TB, _TB)] = r[j]
        if i + _NBUF < nblk:
            start(i + _NBUF)


def _compute_logits_t(x, W):
    n = x.shape[0]
    return pl.pallas_call(
        _logits_body,
        in_specs=[pl.BlockSpec(memory_space=pltpu.HBM),
                  pl.BlockSpec((NUM_EXPERTS, D_MODEL), lambda: (0, 0))],
        out_specs=pl.BlockSpec((NUM_EXPERTS * n,), lambda: (0,)),
        out_shape=jax.ShapeDtypeStruct((NUM_EXPERTS * n,), jnp.float32),
        scratch_shapes=(
            [pltpu.VMEM((_TB, D_MODEL), jnp.float32) for _ in range(_NBUF)]
            + [pltpu.SemaphoreType.DMA((_NBUF,))]
        ),
    )(x, W)


@functools.lru_cache(maxsize=None)
def _make_router(n):
    info = plsc.get_sparse_core_info()
    nc, ns, lanes = info.num_cores, info.num_subcores, info.num_lanes
    nw = nc * ns
    tpw = n // nw
    mesh = plsc.VectorSubcoreMesh(core_axis_name="c", subcore_axis_name="s")

    @functools.partial(
        pl.kernel, mesh=mesh,
        compiler_params=pltpu.CompilerParams(
            needs_layout_passes=False, use_tc_tiling_on_sc=False),
        out_type=[
            jax.ShapeDtypeStruct((TOP_K * n,), jnp.float32),
            jax.ShapeDtypeStruct((TOP_K * n,), jnp.int32),
            jax.ShapeDtypeStruct((MASK_W * n,), jnp.float32),
        ],
        scratch_types=(
            [pltpu.VMEM((NUM_EXPERTS * tpw,), jnp.float32)]
            + [pltpu.VMEM((TOP_K * tpw,), jnp.float32)]
            + [pltpu.VMEM((TOP_K * tpw,), jnp.int32)]
            + [pltpu.VMEM((NUM_EXPERTS * tpw,), jnp.float32) for _ in range(TOP_K)]
            + [pltpu.SemaphoreType.DMA]
        ),
    )
    def router(logits_hbm, probs_hbm, idx_hbm, mask_hbm, *scratch):
        e_v = scratch[0]
        p_v = scratch[1]
        ix_v = scratch[2]
        m_v = scratch[3:3 + TOP_K]
        sem = scratch[-1]
        wid = lax.axis_index("s") * nc + lax.axis_index("c")
        base = wid * tpw
        pltpu.async_copy(
            logits_hbm.at[pl.ds(NUM_EXPERTS * base, NUM_EXPERTS * tpw)],
            e_v, sem).wait()

        @plsc.parallel_loop(0, tpw // lanes, unroll=2)
        def chunk(i):
            sl = pl.ds(i * lanes, lanes)
            e = [e_v[pl.ds(j * tpw + i * lanes, lanes)]
                 for j in range(NUM_EXPERTS)]
            m1 = e[0]
            i1 = jnp.zeros((lanes,), jnp.int32)
            for j in range(1, NUM_EXPERTS):
                gt = e[j] > m1
                m1 = jnp.where(gt, e[j], m1)
                i1 = jnp.where(gt, j, i1)
            m2 = jnp.full((lanes,), -3e38, jnp.float32)
            i2 = jnp.zeros((lanes,), jnp.int32)
            for j in range(NUM_EXPERTS):
                gt = (e[j] > m2) & (i1 != j)
                m2 = jnp.where(gt, e[j], m2)
                i2 = jnp.where(gt, j, i2)
            d = jnp.exp(m2 - m1)
            p1 = 1.0 / (1.0 + d)
            p2 = d * p1
            lb = (i // 8) * 256
            lo = (i % 8) * lanes
            p_v[pl.ds(lb + lo, lanes)] = p1
            p_v[pl.ds(lb + 128 + lo, lanes)] = p2
            ix_v[pl.ds(lb + lo, lanes)] = i1
            ix_v[pl.ds(lb + 128 + lo, lanes)] = i2
            mb = (i // 8) * 1024
            for k in range(NUM_EXPERTS):
                m_v[0][pl.ds(mb + k * 128 + lo, lanes)] = jnp.where(i1 == k, 1.0, 0.0)
                m_v[1][pl.ds(mb + k * 128 + lo, lanes)] = jnp.where(i2 == k, 1.0, 0.0)

        out = [
            pltpu.async_copy(
                p_v, probs_hbm.at[pl.ds(TOP_K * base, TOP_K * tpw)], sem),
            pltpu.async_copy(
                ix_v, idx_hbm.at[pl.ds(TOP_K * base, TOP_K * tpw)], sem),
        ]
        for r in range(TOP_K):
            out.append(pltpu.async_copy(
                m_v[r],
                mask_hbm.at[pl.ds(r * NUM_EXPERTS * n + NUM_EXPERTS * base,
                                  NUM_EXPERTS * tpw)], sem))
        for c in out:
            c.wait()

    return router


def kernel(x, W):
    n = x.shape[0]
    nb = n // 128
    logits_t = _compute_logits_t(x, W)
    probs_t, idx_t, mask_t = _make_router(n)(logits_t)
    probs = probs_t.reshape(nb, TOP_K, 128).transpose(0, 2, 1).reshape(n, TOP_K)
    idx = idx_t.reshape(nb, TOP_K, 128).transpose(0, 2, 1).reshape(n, TOP_K)
    mask = mask_t.reshape(TOP_K, nb, NUM_EXPERTS, 128).transpose(
        1, 3, 0, 2).reshape(n, TOP_K, NUM_EXPERTS)
    return probs, idx, mask


# speedup vs baseline: 2.2623x; 1.0024x over previous
import functools

import jax
import jax.numpy as jnp
from jax import lax
from jax.experimental import pallas as pl
from jax.experimental.pallas import tpu as pltpu
from jax.experimental.pallas import tpu_sc as plsc

D_MODEL = 768
NUM_EXPERTS = 8
TOP_K = 2
MASK_W = TOP_K * NUM_EXPERTS


_TB = 1024
_NBUF = 8


def _logits_body(x_hbm, w_ref, out_ref, *scratch):
    n = x_hbm.shape[0]
    nblk = n // _TB
    bufs = scratch[:_NBUF]
    sems = scratch[_NBUF]

    def start(i):
        pltpu.make_async_copy(
            x_hbm.at[pl.ds(i * _TB, _TB), :],
            bufs[i % _NBUF], sems.at[i % _NBUF]).start()

    for i in range(_NBUF):
        start(i)
    w = w_ref[...]
    for i in range(nblk):
        b = i % _NBUF
        pltpu.make_async_copy(
            x_hbm.at[pl.ds(i * _TB, _TB), :], bufs[b], sems.at[b]).wait()
        r = lax.dot_general(
            w, bufs[b][...],
            dimension_numbers=(((1,), (1,)), ((), ())),
            preferred_element_type=jnp.float32)
        for j in range(NUM_EXPERTS):
            out_ref[pl.ds(i * NUM_EXPERTS * _TB + j * _TB, _TB)] = r[j]
        if i + _NBUF < nblk:
            start(i + _NBUF)


def _compute_logits_t(x, W):
    n = x.shape[0]
    return pl.pallas_call(
        _logits_body,
        in_specs=[pl.BlockSpec(memory_space=pltpu.HBM),
                  pl.BlockSpec((NUM_EXPERTS, D_MODEL), lambda: (0, 0))],
        out_specs=pl.BlockSpec((NUM_EXPERTS * n,), lambda: (0,)),
        out_shape=jax.ShapeDtypeStruct((NUM_EXPERTS * n,), jnp.float32),
        scratch_shapes=(
            [pltpu.VMEM((_TB, D_MODEL), jnp.float32) for _ in range(_NBUF)]
            + [pltpu.SemaphoreType.DMA((_NBUF,))]
        ),
    )(x, W)


@functools.lru_cache(maxsize=None)
def _make_router(n):
    info = plsc.get_sparse_core_info()
    nc, ns, lanes = info.num_cores, info.num_subcores, info.num_lanes
    nw = nc * ns
    tpw = n // nw
    mesh = plsc.VectorSubcoreMesh(core_axis_name="c", subcore_axis_name="s")

    @functools.partial(
        pl.kernel, mesh=mesh,
        compiler_params=pltpu.CompilerParams(
            needs_layout_passes=False, use_tc_tiling_on_sc=False),
        out_type=[
            jax.ShapeDtypeStruct((TOP_K * n,), jnp.float32),
            jax.ShapeDtypeStruct((TOP_K * n,), jnp.int32),
            jax.ShapeDtypeStruct((MASK_W * n,), jnp.float32),
        ],
        scratch_types=(
            [pltpu.VMEM((NUM_EXPERTS * tpw,), jnp.float32)]
            + [pltpu.VMEM((TOP_K * tpw,), jnp.float32)]
            + [pltpu.VMEM((TOP_K * tpw,), jnp.int32)]
            + [pltpu.VMEM((NUM_EXPERTS * tpw,), jnp.float32) for _ in range(TOP_K)]
            + [pltpu.SemaphoreType.DMA]
        ),
    )
    def router(logits_hbm, probs_hbm, idx_hbm, mask_hbm, *scratch):
        e_v = scratch[0]
        p_v = scratch[1]
        ix_v = scratch[2]
        m_v = scratch[3:3 + TOP_K]
        sem = scratch[-1]
        wid = lax.axis_index("s") * nc + lax.axis_index("c")
        base = wid * tpw
        pltpu.async_copy(
            logits_hbm.at[pl.ds(NUM_EXPERTS * base, NUM_EXPERTS * tpw)],
            e_v, sem).wait()

        @plsc.parallel_loop(0, tpw // lanes, unroll=2)
        def chunk(i):
            e = [e_v[pl.ds(j * tpw + i * lanes, lanes)]
                 for j in range(NUM_EXPERTS)]
            m1 = e[0]
            i1 = jnp.zeros((lanes,), jnp.int32)
            for j in range(1, NUM_EXPERTS):
                gt = e[j] > m1
                m1 = jnp.where(gt, e[j], m1)
                i1 = jnp.where(gt, j, i1)
            m2 = jnp.full((lanes,), -3e38, jnp.float32)
            i2 = jnp.zeros((lanes,), jnp.int32)
            for j in range(NUM_EXPERTS):
                gt = (e[j] > m2) & (i1 != j)
                m2 = jnp.where(gt, e[j], m2)
                i2 = jnp.where(gt, j, i2)
            d = jnp.exp(m2 - m1)
            p1 = 1.0 / (1.0 + d)
            p2 = d * p1
            lb = (i // 8) * 256
            lo = (i % 8) * lanes
            p_v[pl.ds(lb + lo, lanes)] = p1
            p_v[pl.ds(lb + 128 + lo, lanes)] = p2
            ix_v[pl.ds(lb + lo, lanes)] = i1
            ix_v[pl.ds(lb + 128 + lo, lanes)] = i2
            mb = (i // 8) * 1024
            for k in range(NUM_EXPERTS):
                m_v[0][pl.ds(mb + k * 128 + lo, lanes)] = jnp.where(i1 == k, 1.0, 0.0)
                m_v[1][pl.ds(mb + k * 128 + lo, lanes)] = jnp.where(i2 == k, 1.0, 0.0)

        out = [
            pltpu.async_copy(
                p_v, probs_hbm.at[pl.ds(TOP_K * base, TOP_K * tpw)], sem),
            pltpu.async_copy(
                ix_v, idx_hbm.at[pl.ds(TOP_K * base, TOP_K * tpw)], sem),
        ]
        for r in range(TOP_K):
            out.append(pltpu.async_copy(
                m_v[r],
                mask_hbm.at[pl.ds(r * NUM_EXPERTS * n + NUM_EXPERTS * base,
                                  NUM_EXPERTS * tpw)], sem))
        for c in out:
            c.wait()

    return router


def kernel(x, W):
    n = x.shape[0]
    nb = n // 128
    logits_t = _compute_logits_t(x, W)
    probs_t, idx_t, mask_t = _make_router(n)(logits_t)
    probs = probs_t.reshape(nb, TOP_K, 128).transpose(0, 2, 1).reshape(n, TOP_K)
    idx = idx_t.reshape(nb, TOP_K, 128).transpose(0, 2, 1).reshape(n, TOP_K)
    mask = mask_t.reshape(TOP_K, nb, NUM_EXPERTS, 128).transpose(
        1, 3, 0, 2).reshape(n, TOP_K, NUM_EXPERTS)
    return probs, idx, mask
